# baseline (speedup 1.0000x reference)
import functools

import jax
import jax.numpy as jnp
from jax.experimental import pallas as pl
from jax.experimental.pallas import tpu as pltpu

_EMA = 2.0 / (2000 + 1)
_T = 1.0
_C = 1.4426950408889634 / _T
_STORE_FRAC = 0.44


def _exp_tile(a, qaug):
    a2 = jnp.sum(a * a, axis=1, keepdims=True)
    aug = jnp.concatenate([a, a2, jnp.ones_like(a2)], axis=1)
    d2 = jnp.dot(aug, qaug,
                 preferred_element_type=jnp.float32)
    d2 = jnp.maximum(d2, 1e-12)
    return jnp.exp2((-_C * d2) * jax.lax.rsqrt(d2))


def _stats_body(qaug_ref, a_ref, z_ref, s_ref, *, nsa):
    i = pl.program_id(0)
    s = _exp_tile(a_ref[...], qaug_ref[...])
    s16 = s.astype(jnp.bfloat16)

    @pl.when(i < nsa)
    def _store():
        s_ref[...] = s16

    ones_row = jnp.ones((1, s.shape[0]), jnp.bfloat16)
    part = jnp.dot(ones_row, s16,
                   preferred_element_type=jnp.float32)

    @pl.when(i == 0)
    def _init():
        z_ref[...] = part

    @pl.when(i > 0)
    def _acc():
        z_ref[...] += part


def _update_body(qall_ref, qaug_ref, s_ref, a_ref, mc_ref, out_ref, p_ref,
                 *, d, nc, nsa):
    i = pl.program_id(0)
    a = a_ref[...]

    @pl.when(i < nsa)
    def _stored():
        p_ref[...] = jnp.dot(s_ref[...], qall_ref[...],
                             preferred_element_type=jnp.float32)

    @pl.when(i >= nsa)
    def _recompute():
        s16 = _exp_tile(a, qaug_ref[...]).astype(jnp.bfloat16)
        p_ref[...] = jnp.dot(s16, qall_ref[...],
                             preferred_element_type=jnp.float32)

    p = p_ref[...]
    wsum = p[:, d + nc:]
    scale = 1.0 - wsum
    am = jnp.concatenate([a, mc_ref[...]], axis=1)
    out_ref[...] = am * scale + p[:, :d + nc]


@jax.jit
def kernel(query_address, query_content, A, Mc):
    b, d = query_address.shape
    m = A.shape[0]
    nc = query_content.shape[1]

    q2 = jnp.sum(query_address * query_address, axis=1)[None, :]
    qaug = jnp.concatenate(
        [-2.0 * query_address.T, jnp.ones((1, b), jnp.float32), q2], axis=0)
    qall = jnp.concatenate(
        [query_address, query_content, jnp.ones((b, 1), jnp.float32)], axis=1)

    mt = 2000 if m % 2000 == 0 else (1000 if m % 1000 == 0 else m)
    nt = m // mt
    nsa = max(1, min(nt, int(round(nt * _STORE_FRAC))))

    full = lambda shape: pl.BlockSpec(shape, lambda i: (0, 0))
    clamp = lambda i: (jnp.minimum(i, nsa - 1), 0)
    z, s16 = pl.pallas_call(
        functools.partial(_stats_body, nsa=nsa),
        grid=(nt,),
        in_specs=[full((d + 2, b)),
                  pl.BlockSpec((mt, d), lambda i: (i, 0))],
        out_specs=[full((1, b)), pl.BlockSpec((mt, b), clamp)],
        out_shape=[jax.ShapeDtypeStruct((1, b), jnp.float32),
                   jax.ShapeDtypeStruct((nsa * mt, b), jnp.bfloat16)],
    )(qaug, A)

    qall_scaled = (qall * (_EMA / z[0])[:, None]).astype(jnp.bfloat16)

    out = pl.pallas_call(
        functools.partial(_update_body, d=d, nc=nc, nsa=nsa),
        grid=(nt,),
        in_specs=[full((b, d + nc + 1)), full((d + 2, b)),
                  pl.BlockSpec((mt, b), clamp),
                  pl.BlockSpec((mt, d), lambda i: (i, 0)),
                  pl.BlockSpec((mt, nc), lambda i: (i, 0))],
        out_specs=pl.BlockSpec((mt, d + nc), lambda i: (i, 0)),
        out_shape=jax.ShapeDtypeStruct((m, d + nc), jnp.float32),
        scratch_shapes=[pltpu.VMEM((mt, d + nc + 1), jnp.float32)],
    )(qall_scaled, qaug, s16, A, Mc)
    return out


# speedup vs baseline: 1.1867x; 1.1867x over previous
import functools

import jax
import jax.numpy as jnp
from jax.experimental import pallas as pl

_EMA = 2.0 / (2000 + 1)
_T = 1.0
_C = 1.4426950408889634 / _T
_STORE_ROWS = 880


def _exp_tile(a, qaug):
    a2 = jnp.sum(a * a, axis=1, keepdims=True)
    aug = jnp.concatenate([a, a2, jnp.ones_like(a2)], axis=1)
    d2 = jnp.dot(aug, qaug,
                 preferred_element_type=jnp.float32)
    d2 = jnp.maximum(d2, 1e-12)
    return jnp.exp2((-_C * d2) * jax.lax.rsqrt(d2))


def _stats_body(qaug_ref, a_ref, z_ref, s_ref, *, ms):
    i = pl.program_id(0)
    s = _exp_tile(a_ref[...], qaug_ref[...])
    s16 = s.astype(jnp.bfloat16)
    s_ref[...] = s16[:ms]
    ones_row = jnp.ones((1, s.shape[0]), jnp.bfloat16)
    part = jnp.dot(ones_row, s16,
                   preferred_element_type=jnp.float32)

    @pl.when(i == 0)
    def _init():
        z_ref[...] = part

    @pl.when(i > 0)
    def _acc():
        z_ref[...] += part


def _update_body(qall_ref, qaug_ref, s_ref, a_ref, mc_ref, out_ref,
                 *, d, nc, ms):
    a = a_ref[...]
    qall = qall_ref[...]
    p_top = jnp.dot(s_ref[...], qall,
                    preferred_element_type=jnp.float32)
    s_bot = _exp_tile(a[ms:], qaug_ref[...]).astype(jnp.bfloat16)
    p_bot = jnp.dot(s_bot, qall,
                    preferred_element_type=jnp.float32)
    am = jnp.concatenate([a, mc_ref[...]], axis=1)
    out_ref[:ms] = am[:ms] * (1.0 - p_top[:, d + nc:]) + p_top[:, :d + nc]
    out_ref[ms:] = am[ms:] * (1.0 - p_bot[:, d + nc:]) + p_bot[:, :d + nc]


@jax.jit
def kernel(query_address, query_content, A, Mc):
    b, d = query_address.shape
    m = A.shape[0]
    nc = query_content.shape[1]

    q2 = jnp.sum(query_address * query_address, axis=1)[None, :]
    qaug = jnp.concatenate(
        [-2.0 * query_address.T, jnp.ones((1, b), jnp.float32), q2], axis=0)
    qall = jnp.concatenate(
        [query_address, query_content, jnp.ones((b, 1), jnp.float32)], axis=1)

    mt = 2000 if m % 2000 == 0 else (1000 if m % 1000 == 0 else m)
    nt = m // mt
    ms = _STORE_ROWS if mt == 2000 else max(8, (mt * 44 // 100) // 8 * 8)

    full = lambda shape: pl.BlockSpec(shape, lambda i: (0, 0))
    z, s16 = pl.pallas_call(
        functools.partial(_stats_body, ms=ms),
        grid=(nt,),
        in_specs=[full((d + 2, b)),
                  pl.BlockSpec((mt, d), lambda i: (i, 0))],
        out_specs=[full((1, b)), pl.BlockSpec((ms, b), lambda i: (i, 0))],
        out_shape=[jax.ShapeDtypeStruct((1, b), jnp.float32),
                   jax.ShapeDtypeStruct((nt * ms, b), jnp.bfloat16)],
    )(qaug, A)

    qall_scaled = (qall * (_EMA / z[0])[:, None]).astype(jnp.bfloat16)

    out = pl.pallas_call(
        functools.partial(_update_body, d=d, nc=nc, ms=ms),
        grid=(nt,),
        in_specs=[full((b, d + nc + 1)), full((d + 2, b)),
                  pl.BlockSpec((ms, b), lambda i: (i, 0)),
                  pl.BlockSpec((mt, d), lambda i: (i, 0)),
                  pl.BlockSpec((mt, nc), lambda i: (i, 0))],
        out_specs=pl.BlockSpec((mt, d + nc), lambda i: (i, 0)),
        out_shape=jax.ShapeDtypeStruct((m, d + nc), jnp.float32),
    )(qall_scaled, qaug, s16, A, Mc)
    return out
